# Initial kernel scaffold; baseline (speedup 1.0000x reference)
import jax
import jax.numpy as jnp
from jax.experimental import pallas as pl


def kernel(logits, labels):
    raise NotImplementedError("write your pallas kernel here")


# speedup vs baseline: 3.2222x; 3.2222x over previous
import math

import jax
import jax.numpy as jnp
from jax.experimental import pallas as pl

_S = 64.0
_M2 = 0.5
_IGNORE = 98

_BN = 2048


def _margin_body(labels_ref, x_ref, o_ref):
    j = pl.program_id(0)
    x = x_ref[...]
    lab = labels_ref[...]
    bn = x.shape[1]
    cols = jax.lax.broadcasted_iota(jnp.int32, x.shape, 1) + j * bn
    hit = lab == cols
    g = jnp.sum(jnp.where(hit, x, 0.0), axis=1, keepdims=True)
    cm, sm = math.cos(_M2), math.sin(_M2)
    adj = g * jnp.float32(cm) - jnp.sqrt(jnp.maximum(1.0 - g * g, 0.0)) * jnp.float32(sm)
    fixed = jnp.where(lab != _IGNORE, adj, g)
    o_ref[...] = jnp.float32(_S) * jnp.where(hit, fixed, x)


@jax.jit
def kernel(logits, labels):
    B, V = logits.shape
    labels2d = labels.astype(jnp.int32).reshape(B, 1)
    grid = (pl.cdiv(V, _BN),)
    return pl.pallas_call(
        _margin_body,
        grid=grid,
        in_specs=[
            pl.BlockSpec((B, 1), lambda j: (0, 0)),
            pl.BlockSpec((B, _BN), lambda j: (0, j)),
        ],
        out_specs=pl.BlockSpec((B, _BN), lambda j: (0, j)),
        out_shape=jax.ShapeDtypeStruct((B, V), jnp.float32),
    )(labels2d, logits)
